# baseline (speedup 1.0000x reference)
import functools

import jax
import jax.numpy as jnp
from jax import lax
from jax.experimental import pallas as pl
from jax.experimental.pallas import tpu as pltpu
from jax.experimental.pallas import tpu_sc as plsc

VOCAB = 100000
BATCH = 128
R = 64
K = 2048
LAM = 0.5

NC = 2
NS = 16
NW = NC * NS
RULES_PER_W = R // NW
IDX_PER_W = RULES_PER_W * K

V_PAD = 100352
SLICE = V_PAD // NS
BB = 8
GRID_B = BATCH // BB

def _sc_hist_body(vio_hbm, gl_hbm, out_hbm, idx_v, vals_v, gl_v, stage_v, hist_sh):
    cid = lax.axis_index("c")
    sid = lax.axis_index("s")
    wid = cid * NS + sid
    base = sid * SLICE

    zv = jnp.zeros((16,), jnp.float32)

    def _zero(i, carry):
        stage_v[pl.ds(i * 16, 16)] = zv
        return carry

    lax.fori_loop(0, SLICE // 16, _zero, 0)
    pltpu.sync_copy(stage_v, hist_sh.at[pl.ds(base, SLICE)])

    pltpu.sync_copy(vio_hbm.at[pl.ds(wid * IDX_PER_W, IDX_PER_W)], idx_v)
    pltpu.sync_copy(gl_hbm, gl_v)

    gvecs = []
    for b in range(R // 16):
        glv = gl_v[pl.ds(b * 16, 16)]
        gvecs.append(LAM / (1.0 + jnp.exp(-glv)))

    for j in range(RULES_PER_W):
        r = wid * RULES_PER_W + j
        blk = r // 16
        gsel = gvecs[0]
        for b in range(1, R // 16):
            gsel = jnp.where(blk == b, gvecs[b], gsel)
        lane = jnp.full((16,), r % 16, jnp.int32)
        vec = jnp.take_along_axis(gsel, lane, axis=0)

        def _fill(i, carry, _j=j, _vec=vec):
            vals_v[pl.ds(_j * K + i * 16, 16)] = _vec
            return carry

        lax.fori_loop(0, K // 16, _fill, 0)

    plsc.subcore_barrier()
    pltpu.sync_copy(vals_v, hist_sh.at[idx_v], add=True)
    plsc.subcore_barrier()
    pltpu.sync_copy(hist_sh.at[pl.ds(base, SLICE)], out_hbm.at[cid, sid])


@functools.lru_cache(maxsize=None)
def _make_sc_hist():
    mesh = plsc.VectorSubcoreMesh(core_axis_name="c", subcore_axis_name="s")
    return pl.kernel(
        _sc_hist_body,
        out_type=jax.ShapeDtypeStruct((NC, NS, SLICE), jnp.float32),
        mesh=mesh,
        scratch_types=[
            pltpu.VMEM((IDX_PER_W,), jnp.int32),
            pltpu.VMEM((IDX_PER_W,), jnp.float32),
            pltpu.VMEM((R,), jnp.float32),
            pltpu.VMEM((SLICE,), jnp.float32),
            pltpu.VMEM_SHARED((V_PAD,), jnp.float32),
        ],
    )


def _tc_body(logits_ref, pt_ref, gl_ref, mod_ref, pen_ref, cov_ref):
    pt = pt_ref[0:1, :VOCAB] + pt_ref[1:2, :VOCAB]
    pen = jnp.broadcast_to(pt, (BB, VOCAB))
    pen_ref[...] = pen
    mod_ref[...] = logits_ref[...] - pen

    @pl.when(pl.program_id(0) == 0)
    def _():
        g = jax.nn.sigmoid(gl_ref[0, :])
        cov_ref[0, 0] = -jnp.sum(g) / R


_tc_call = pl.pallas_call(
    _tc_body,
    grid=(GRID_B,),
    in_specs=[
        pl.BlockSpec((BB, VOCAB), lambda i: (i, 0)),
        pl.BlockSpec((NC, V_PAD), lambda i: (0, 0)),
        pl.BlockSpec((1, R), lambda i: (0, 0)),
    ],
    out_specs=[
        pl.BlockSpec((BB, VOCAB), lambda i: (i, 0)),
        pl.BlockSpec((BB, VOCAB), lambda i: (i, 0)),
        pl.BlockSpec(memory_space=pltpu.SMEM),
    ],
    out_shape=[
        jax.ShapeDtypeStruct((BATCH, VOCAB), jnp.float32),
        jax.ShapeDtypeStruct((BATCH, VOCAB), jnp.float32),
        jax.ShapeDtypeStruct((1, 1), jnp.float32),
    ],
    compiler_params=pltpu.CompilerParams(
        dimension_semantics=("arbitrary",),
    ),
)


def kernel(logits, violation_indices_per_rule, gate_logits):
    vio = violation_indices_per_rule.astype(jnp.int32).reshape(R * K)
    gl = gate_logits.astype(jnp.float32)

    partial = _make_sc_hist()(vio, gl)
    pt = partial.reshape(NC, V_PAD)


    modified, penalties, cov = _tc_call(logits, pt, gl.reshape(1, R))
    coverage_loss = cov.reshape(())
    return modified, coverage_loss, penalties


# speedup vs baseline: 1.0128x; 1.0128x over previous
import functools

import jax
import jax.numpy as jnp
from jax import lax
from jax.experimental import pallas as pl
from jax.experimental.pallas import tpu as pltpu
from jax.experimental.pallas import tpu_sc as plsc

VOCAB = 100000
BATCH = 128
R = 64
K = 2048
LAM = 0.5

NC = 2
NS = 16
NW = NC * NS
RULES_PER_W = R // NW
IDX_PER_W = RULES_PER_W * K

V_PAD = 100352
SLICE = V_PAD // NS
BB = 16
GRID_B = BATCH // BB

def _sc_hist_body(vio_hbm, gl_hbm, out_hbm, idx_v, vals_v, gl_v, stage_v, hist_sh):
    cid = lax.axis_index("c")
    sid = lax.axis_index("s")
    wid = cid * NS + sid
    base = sid * SLICE

    zv = jnp.zeros((16,), jnp.float32)

    def _zero(i, carry):
        stage_v[pl.ds(i * 16, 16)] = zv
        return carry

    lax.fori_loop(0, SLICE // 16, _zero, 0)
    pltpu.sync_copy(stage_v, hist_sh.at[pl.ds(base, SLICE)])

    pltpu.sync_copy(vio_hbm.at[pl.ds(wid * IDX_PER_W, IDX_PER_W)], idx_v)
    pltpu.sync_copy(gl_hbm, gl_v)

    gvecs = []
    for b in range(R // 16):
        glv = gl_v[pl.ds(b * 16, 16)]
        gvecs.append(LAM / (1.0 + jnp.exp(-glv)))

    for j in range(RULES_PER_W):
        r = wid * RULES_PER_W + j
        blk = r // 16
        gsel = gvecs[0]
        for b in range(1, R // 16):
            gsel = jnp.where(blk == b, gvecs[b], gsel)
        lane = jnp.full((16,), r % 16, jnp.int32)
        vec = jnp.take_along_axis(gsel, lane, axis=0)

        def _fill(i, carry, _j=j, _vec=vec):
            vals_v[pl.ds(_j * K + i * 16, 16)] = _vec
            return carry

        lax.fori_loop(0, K // 16, _fill, 0)

    plsc.subcore_barrier()
    pltpu.sync_copy(vals_v, hist_sh.at[idx_v], add=True)
    plsc.subcore_barrier()
    pltpu.sync_copy(hist_sh.at[pl.ds(base, SLICE)], out_hbm.at[cid, sid])


@functools.lru_cache(maxsize=None)
def _make_sc_hist():
    mesh = plsc.VectorSubcoreMesh(core_axis_name="c", subcore_axis_name="s")
    return pl.kernel(
        _sc_hist_body,
        out_type=jax.ShapeDtypeStruct((NC, NS, SLICE), jnp.float32),
        mesh=mesh,
        scratch_types=[
            pltpu.VMEM((IDX_PER_W,), jnp.int32),
            pltpu.VMEM((IDX_PER_W,), jnp.float32),
            pltpu.VMEM((R,), jnp.float32),
            pltpu.VMEM((SLICE,), jnp.float32),
            pltpu.VMEM_SHARED((V_PAD,), jnp.float32),
        ],
    )


def _tc_body(logits_ref, pt_ref, gl_ref, mod_ref, pen_ref, cov_ref):
    pt = pt_ref[0:1, :VOCAB] + pt_ref[1:2, :VOCAB]
    pen = jnp.broadcast_to(pt, (BB, VOCAB))
    pen_ref[...] = pen
    mod_ref[...] = logits_ref[...] - pen

    @pl.when(pl.program_id(0) == 0)
    def _():
        g = jax.nn.sigmoid(gl_ref[0, :])
        cov_ref[0, 0] = -jnp.sum(g) / R


_tc_call = pl.pallas_call(
    _tc_body,
    grid=(GRID_B,),
    in_specs=[
        pl.BlockSpec((BB, VOCAB), lambda i: (i, 0)),
        pl.BlockSpec((NC, V_PAD), lambda i: (0, 0)),
        pl.BlockSpec((1, R), lambda i: (0, 0)),
    ],
    out_specs=[
        pl.BlockSpec((BB, VOCAB), lambda i: (i, 0)),
        pl.BlockSpec((BB, VOCAB), lambda i: (i, 0)),
        pl.BlockSpec(memory_space=pltpu.SMEM),
    ],
    out_shape=[
        jax.ShapeDtypeStruct((BATCH, VOCAB), jnp.float32),
        jax.ShapeDtypeStruct((BATCH, VOCAB), jnp.float32),
        jax.ShapeDtypeStruct((1, 1), jnp.float32),
    ],
    compiler_params=pltpu.CompilerParams(
        dimension_semantics=("arbitrary",),
    ),
)


def kernel(logits, violation_indices_per_rule, gate_logits):
    vio = violation_indices_per_rule.astype(jnp.int32).reshape(R * K)
    gl = gate_logits.astype(jnp.float32)

    partial = _make_sc_hist()(vio, gl)
    pt = partial.reshape(NC, V_PAD)


    modified, penalties, cov = _tc_call(logits, pt, gl.reshape(1, R))
    coverage_loss = cov.reshape(())
    return modified, coverage_loss, penalties


# speedup vs baseline: 1.3631x; 1.3459x over previous
import functools

import jax
import jax.numpy as jnp
from jax import lax
from jax.experimental import pallas as pl
from jax.experimental.pallas import tpu as pltpu
from jax.experimental.pallas import tpu_sc as plsc

VOCAB = 100000
BATCH = 128
R = 64
K = 2048
LAM = 0.5

NC = 2
NS = 16
NW = NC * NS
RULES_PER_W = R // NW
IDX_PER_W = RULES_PER_W * K

V_PAD = 100352
SLICE = V_PAD // NS
BVR = 10000
GRID_VR = VOCAB // BVR

def _sc_hist_body(vio_hbm, gl_hbm, out_hbm, idx_v, vals_v, gl_v, stage_v, hist_sh):
    cid = lax.axis_index("c")
    sid = lax.axis_index("s")
    wid = cid * NS + sid
    base = sid * SLICE

    zv = jnp.zeros((16,), jnp.float32)

    def _zero(i, carry):
        stage_v[pl.ds(i * 16, 16)] = zv
        return carry

    lax.fori_loop(0, SLICE // 16, _zero, 0)
    pltpu.sync_copy(stage_v, hist_sh.at[pl.ds(base, SLICE)])

    pltpu.sync_copy(vio_hbm.at[pl.ds(wid * IDX_PER_W, IDX_PER_W)], idx_v)
    pltpu.sync_copy(gl_hbm, gl_v)

    gvecs = []
    for b in range(R // 16):
        glv = gl_v[pl.ds(b * 16, 16)]
        gvecs.append(LAM / (1.0 + jnp.exp(-glv)))

    for j in range(RULES_PER_W):
        r = wid * RULES_PER_W + j
        blk = r // 16
        gsel = gvecs[0]
        for b in range(1, R // 16):
            gsel = jnp.where(blk == b, gvecs[b], gsel)
        lane = jnp.full((16,), r % 16, jnp.int32)
        vec = jnp.take_along_axis(gsel, lane, axis=0)

        def _fill(i, carry, _j=j, _vec=vec):
            vals_v[pl.ds(_j * K + i * 16, 16)] = _vec
            return carry

        lax.fori_loop(0, K // 16, _fill, 0)

    plsc.subcore_barrier()
    pltpu.sync_copy(vals_v, hist_sh.at[idx_v], add=True)
    plsc.subcore_barrier()
    pltpu.sync_copy(hist_sh.at[pl.ds(base, SLICE)], out_hbm.at[cid, sid])


@functools.lru_cache(maxsize=None)
def _make_sc_hist():
    mesh = plsc.VectorSubcoreMesh(core_axis_name="c", subcore_axis_name="s")
    return pl.kernel(
        _sc_hist_body,
        out_type=jax.ShapeDtypeStruct((NC, NS, SLICE), jnp.float32),
        mesh=mesh,
        scratch_types=[
            pltpu.VMEM((IDX_PER_W,), jnp.int32),
            pltpu.VMEM((IDX_PER_W,), jnp.float32),
            pltpu.VMEM((R,), jnp.float32),
            pltpu.VMEM((SLICE,), jnp.float32),
            pltpu.VMEM_SHARED((V_PAD,), jnp.float32),
        ],
    )


def _tc_body(lt_ref, pt_ref, gl_ref, mod_ref, pen_ref, cov_ref):
    pen = jnp.broadcast_to(pt_ref[:, 0:1] + pt_ref[:, 1:2], (BVR, BATCH))
    pen_ref[...] = pen
    mod_ref[...] = lt_ref[...] - pen

    @pl.when(pl.program_id(0) == 0)
    def _():
        g = jax.nn.sigmoid(gl_ref[0, :])
        cov_ref[0, 0] = -jnp.sum(g) / R


_tc_call = pl.pallas_call(
    _tc_body,
    grid=(GRID_VR,),
    in_specs=[
        pl.BlockSpec((BVR, BATCH), lambda i: (i, 0)),
        pl.BlockSpec((BVR, NC), lambda i: (i, 0)),
        pl.BlockSpec((1, R), lambda i: (0, 0)),
    ],
    out_specs=[
        pl.BlockSpec((BVR, BATCH), lambda i: (i, 0)),
        pl.BlockSpec((BVR, BATCH), lambda i: (i, 0)),
        pl.BlockSpec(memory_space=pltpu.SMEM),
    ],
    out_shape=[
        jax.ShapeDtypeStruct((VOCAB, BATCH), jnp.float32),
        jax.ShapeDtypeStruct((VOCAB, BATCH), jnp.float32),
        jax.ShapeDtypeStruct((1, 1), jnp.float32),
    ],
    compiler_params=pltpu.CompilerParams(
        dimension_semantics=("arbitrary",),
    ),
)


def kernel(logits, violation_indices_per_rule, gate_logits):
    vio = violation_indices_per_rule.astype(jnp.int32).reshape(R * K)
    gl = gate_logits.astype(jnp.float32)

    partial = _make_sc_hist()(vio, gl)
    pt = partial.reshape(NC, V_PAD)
    ptT = pt[:, :VOCAB].T

    lt = logits.T
    modT, penT, cov = _tc_call(lt, ptT, gl.reshape(1, R))
    coverage_loss = cov.reshape(())
    return modT.T, coverage_loss, penT.T


# speedup vs baseline: 2.6488x; 1.9433x over previous
import functools

import jax
import jax.numpy as jnp
from jax import lax
from jax.experimental import pallas as pl
from jax.experimental.pallas import tpu as pltpu
from jax.experimental.pallas import tpu_sc as plsc

VOCAB = 100000
BATCH = 128
R = 64
K = 2048
LAM = 0.5

NC = 2
NS = 16
NW = NC * NS
RULES_PER_W = R // NW
IDX_PER_W = RULES_PER_W * K

V_PAD = 100352
SLICE = V_PAD // NS
BVR = 12544
GRID_VR = V_PAD // BVR

def _sc_hist_body(vio_hbm, gl_hbm, out_hbm, idx_v, vals_v, gl_v, stage_v, hist_sh):
    cid = lax.axis_index("c")
    sid = lax.axis_index("s")
    wid = cid * NS + sid
    base = sid * SLICE

    zv = jnp.zeros((16,), jnp.float32)

    def _zero(i, carry):
        stage_v[pl.ds(i * 16, 16)] = zv
        return carry

    lax.fori_loop(0, SLICE // 16, _zero, 0)
    pltpu.sync_copy(stage_v, hist_sh.at[pl.ds(base, SLICE)])

    pltpu.sync_copy(vio_hbm.at[pl.ds(wid * IDX_PER_W, IDX_PER_W)], idx_v)
    pltpu.sync_copy(gl_hbm, gl_v)

    gvecs = []
    for b in range(R // 16):
        glv = gl_v[pl.ds(b * 16, 16)]
        gvecs.append(LAM / (1.0 + jnp.exp(-glv)))

    for j in range(RULES_PER_W):
        r = wid * RULES_PER_W + j
        blk = r // 16
        gsel = gvecs[0]
        for b in range(1, R // 16):
            gsel = jnp.where(blk == b, gvecs[b], gsel)
        lane = jnp.full((16,), r % 16, jnp.int32)
        vec = jnp.take_along_axis(gsel, lane, axis=0)

        def _fill(i, carry, _j=j, _vec=vec):
            vals_v[pl.ds(_j * K + i * 16, 16)] = _vec
            return carry

        lax.fori_loop(0, K // 16, _fill, 0)

    plsc.subcore_barrier()
    pltpu.sync_copy(vals_v, hist_sh.at[idx_v], add=True)
    plsc.subcore_barrier()
    pltpu.sync_copy(hist_sh.at[pl.ds(base, SLICE)], out_hbm.at[cid, sid])


@functools.lru_cache(maxsize=None)
def _make_sc_hist():
    mesh = plsc.VectorSubcoreMesh(core_axis_name="c", subcore_axis_name="s")
    return pl.kernel(
        _sc_hist_body,
        out_type=jax.ShapeDtypeStruct((NC, NS, SLICE), jnp.float32),
        mesh=mesh,
        scratch_types=[
            pltpu.VMEM((IDX_PER_W,), jnp.int32),
            pltpu.VMEM((IDX_PER_W,), jnp.float32),
            pltpu.VMEM((R,), jnp.float32),
            pltpu.VMEM((SLICE,), jnp.float32),
            pltpu.VMEM_SHARED((V_PAD,), jnp.float32),
        ],
    )


def _tc_body(lt_ref, pt_ref, gl_ref, mod_ref, pen_ref, cov_ref):
    s = pt_ref[0:1, :] + pt_ref[1:2, :]
    ones = jnp.ones((1, BATCH), jnp.float32)
    pen = lax.dot_general(
        s, ones, (((0,), (0,)), ((), ())),
        preferred_element_type=jnp.float32,
    )
    pen_ref[...] = pen
    mod_ref[...] = lt_ref[...] - pen

    @pl.when(pl.program_id(0) == 0)
    def _():
        g = jax.nn.sigmoid(gl_ref[0, :])
        cov_ref[0, 0] = -jnp.sum(g) / R


_tc_call = pl.pallas_call(
    _tc_body,
    grid=(GRID_VR,),
    in_specs=[
        pl.BlockSpec((BVR, BATCH), lambda i: (i, 0)),
        pl.BlockSpec((NC, BVR), lambda i: (0, i)),
        pl.BlockSpec((1, R), lambda i: (0, 0)),
    ],
    out_specs=[
        pl.BlockSpec((BVR, BATCH), lambda i: (i, 0)),
        pl.BlockSpec((BVR, BATCH), lambda i: (i, 0)),
        pl.BlockSpec(memory_space=pltpu.SMEM),
    ],
    out_shape=[
        jax.ShapeDtypeStruct((VOCAB, BATCH), jnp.float32),
        jax.ShapeDtypeStruct((VOCAB, BATCH), jnp.float32),
        jax.ShapeDtypeStruct((1, 1), jnp.float32),
    ],
    compiler_params=pltpu.CompilerParams(
        dimension_semantics=("arbitrary",),
    ),
)


def kernel(logits, violation_indices_per_rule, gate_logits):
    vio = violation_indices_per_rule.astype(jnp.int32).reshape(R * K)
    gl = gate_logits.astype(jnp.float32)

    partial = _make_sc_hist()(vio, gl)
    pt = partial.reshape(NC, V_PAD)

    lt = logits.T
    modT, penT, cov = _tc_call(lt, pt, gl.reshape(1, R))
    coverage_loss = cov.reshape(())
    return modT.T, coverage_loss, penT.T
